# baseline (speedup 1.0000x reference)
import jax
import jax.numpy as jnp
from jax import lax
from jax.experimental import pallas as pl
from jax.experimental.pallas import tpu as pltpu
from jax.experimental.pallas import tpu_sc as plsc

N_EXPERTS = 16
TOP_K = 2
D_MODEL = 2048
N_TOKENS = 16384

BLK = 1024

NC = 2
NS = 16
NW = NC * NS
TOK_PER_W = N_TOKENS // NW
LANES = 16
TTILES = TOK_PER_W // 128


def _gemm_body(t_ref, x_ref, w_ref, lg_ref):
    inv_t = 1.0 / t_ref[0]
    lg_ref[...] = jax.lax.dot_general(
        w_ref[...], x_ref[...],
        dimension_numbers=(((1,), (1,)), ((), ())),
        preferred_element_type=jnp.float32,
    ) * inv_t


def _gate_logits_t(x, W, t):
    return pl.pallas_call(
        _gemm_body,
        grid=(N_TOKENS // BLK,),
        in_specs=[
            pl.BlockSpec(memory_space=pltpu.SMEM),
            pl.BlockSpec((BLK, D_MODEL), lambda i: (i, 0)),
            pl.BlockSpec((N_EXPERTS, D_MODEL), lambda i: (0, 0)),
        ],
        out_specs=pl.BlockSpec((N_EXPERTS, BLK), lambda i: (0, i)),
        out_shape=jax.ShapeDtypeStruct((N_EXPERTS, N_TOKENS), jnp.float32),
        compiler_params=pltpu.CompilerParams(
            dimension_semantics=("arbitrary",),
        ),
    )(t, x, W)


def _sc_route_body(lgt_hbm, rm_hbm, idx_hbm, lg_v, rm_v, idx_v, sem):
    wid = lax.axis_index("s") * NC + lax.axis_index("c")
    base = wid * TOK_PER_W

    lane = lax.broadcasted_iota(jnp.int32, (LANES,), 0)
    zero_f = jnp.zeros((LANES,), jnp.float32)
    neg_inf = jnp.full((LANES,), -jnp.inf, jnp.float32)
    e_consts = [jnp.full((LANES,), e, jnp.int32) for e in range(N_EXPERTS)]

    def ttile(tt, carry):
        tok0 = base + tt * 128
        cp0 = pltpu.async_copy(
            lgt_hbm.at[pl.ds(0, 8), pl.ds(tok0, 128)],
            lg_v.at[pl.ds(0, 8), :], sem)
        cp1 = pltpu.async_copy(
            lgt_hbm.at[pl.ds(8, 8), pl.ds(tok0, 128)],
            lg_v.at[pl.ds(8, 8), :], sem)
        cp0.wait()
        cp1.wait()

        def group(j, c2):
            cols = [lg_v[e, pl.ds(j * LANES, LANES)] for e in range(N_EXPERTS)]
            m0 = cols[0]
            i0 = e_consts[0]
            for e in range(1, N_EXPERTS):
                gt = cols[e] > m0
                m0 = jnp.where(gt, cols[e], m0)
                i0 = jnp.where(gt, e_consts[e], i0)
            m1 = neg_inf
            i1 = e_consts[0]
            for e in range(N_EXPERTS):
                cand = jnp.where(i0 == e_consts[e], neg_inf, cols[e])
                gt = cand > m1
                m1 = jnp.where(gt, cand, m1)
                i1 = jnp.where(gt, e_consts[e], i1)
            ex = jnp.exp(m1 - m0)
            w0 = 1.0 / (1.0 + ex)
            w1 = ex * w0
            for e in range(N_EXPERTS):
                rm_v[e, pl.ds(j * LANES, LANES)] = zero_f
            tpos = jnp.full((LANES,), j * LANES, jnp.int32) + lane
            plsc.store_scatter(rm_v, [i0, tpos], w0)
            plsc.store_scatter(rm_v, [i1, tpos], w1)
            idx_v[0, pl.ds(j * LANES, LANES)] = i0
            idx_v[1, pl.ds(j * LANES, LANES)] = i1
            return c2

        lax.fori_loop(0, 128 // LANES, group, 0)

        pltpu.sync_copy(rm_v.at[pl.ds(0, 8), :],
                        rm_hbm.at[pl.ds(0, 8), pl.ds(tok0, 128)])
        pltpu.sync_copy(rm_v.at[pl.ds(8, 8), :],
                        rm_hbm.at[pl.ds(8, 8), pl.ds(tok0, 128)])
        pltpu.sync_copy(idx_v, idx_hbm.at[:, pl.ds(tok0, 128)])
        return carry

    lax.fori_loop(0, TTILES, ttile, 0)


def _sc_route(logits_t):
    mesh = plsc.VectorSubcoreMesh(core_axis_name="c", subcore_axis_name="s")
    return pl.kernel(
        _sc_route_body,
        mesh=mesh,
        out_type=[
            jax.ShapeDtypeStruct((N_EXPERTS, N_TOKENS), jnp.float32),
            jax.ShapeDtypeStruct((TOP_K, N_TOKENS), jnp.int32),
        ],
        scratch_types=[
            pltpu.VMEM((N_EXPERTS, 128), jnp.float32),
            pltpu.VMEM((N_EXPERTS, 128), jnp.float32),
            pltpu.VMEM((TOP_K, 128), jnp.int32),
            pltpu.SemaphoreType.DMA,
        ],
        compiler_params=pltpu.CompilerParams(
            needs_layout_passes=False,
            use_tc_tiling_on_sc=True,
        ),
    )(logits_t)


def kernel(x, W, temperature):
    t = jnp.asarray(temperature, jnp.float32).reshape(1)
    logits_t = _gate_logits_t(x, W, t)
    rm_t, idx_t = _sc_route(logits_t)
    return (rm_t.T, idx_t.T)


# speedup vs baseline: 1.0312x; 1.0312x over previous
import jax
import jax.numpy as jnp
from jax import lax
from jax.experimental import pallas as pl
from jax.experimental.pallas import tpu as pltpu
from jax.experimental.pallas import tpu_sc as plsc

N_EXPERTS = 16
TOP_K = 2
D_MODEL = 2048
N_TOKENS = 16384

BLK = 1024

NC = 2
NS = 16
NW = NC * NS
TOK_PER_W = N_TOKENS // NW
LANES = 16
TTILES = TOK_PER_W // 128


def _gemm_body(t_ref, x_ref, w_ref, lg_ref):
    inv_t = 1.0 / t_ref[0]
    lg_ref[...] = jax.lax.dot_general(
        w_ref[...], x_ref[...],
        dimension_numbers=(((1,), (1,)), ((), ())),
        preferred_element_type=jnp.float32,
    ) * inv_t


def _gate_logits_t(x, W, t):
    return pl.pallas_call(
        _gemm_body,
        grid=(N_TOKENS // BLK,),
        in_specs=[
            pl.BlockSpec(memory_space=pltpu.SMEM),
            pl.BlockSpec((BLK, D_MODEL), lambda i: (i, 0)),
            pl.BlockSpec((N_EXPERTS, D_MODEL), lambda i: (0, 0)),
        ],
        out_specs=pl.BlockSpec((N_EXPERTS, BLK), lambda i: (0, i)),
        out_shape=jax.ShapeDtypeStruct((N_EXPERTS, N_TOKENS), jnp.float32),
        compiler_params=pltpu.CompilerParams(
            dimension_semantics=("arbitrary",),
        ),
    )(t, x, W)


def _sc_route_body(lgt_hbm, rm_hbm, idx_hbm, lg_v, rm_v, idx_v, sem):
    wid = lax.axis_index("s") * NC + lax.axis_index("c")
    base = wid * TOK_PER_W

    neg_inf = jnp.full((LANES,), -jnp.inf, jnp.float32)
    zero_f = jnp.zeros((LANES,), jnp.float32)
    e_consts = [jnp.full((LANES,), e, jnp.int32) for e in range(N_EXPERTS)]

    cp0 = pltpu.async_copy(
        lgt_hbm.at[pl.ds(0, 8), pl.ds(base, TOK_PER_W)],
        lg_v.at[pl.ds(0, 8), :], sem)
    cp1 = pltpu.async_copy(
        lgt_hbm.at[pl.ds(8, 8), pl.ds(base, TOK_PER_W)],
        lg_v.at[pl.ds(8, 8), :], sem)
    cp0.wait()
    cp1.wait()

    def group(j, c2):
        o = j * LANES
        cols = [lg_v[e, pl.ds(o, LANES)] for e in range(N_EXPERTS)]
        m0 = cols[0]
        i0 = e_consts[0]
        for e in range(1, N_EXPERTS):
            gt = cols[e] > m0
            m0 = jnp.where(gt, cols[e], m0)
            i0 = jnp.where(gt, e_consts[e], i0)
        m1 = neg_inf
        i1 = e_consts[0]
        for e in range(N_EXPERTS):
            cand = jnp.where(i0 == e_consts[e], neg_inf, cols[e])
            gt = cand > m1
            m1 = jnp.where(gt, cand, m1)
            i1 = jnp.where(gt, e_consts[e], i1)
        ex = jnp.exp(m1 - m0)
        w0 = 1.0 / (1.0 + ex)
        w1 = ex * w0
        for e in range(N_EXPERTS):
            rm_v[e, pl.ds(o, LANES)] = jnp.where(
                i0 == e_consts[e], w0,
                jnp.where(i1 == e_consts[e], w1, zero_f))
        idx_v[0, pl.ds(o, LANES)] = i0
        idx_v[1, pl.ds(o, LANES)] = i1
        return c2

    lax.fori_loop(0, TOK_PER_W // LANES, group, 0)

    pltpu.sync_copy(rm_v.at[pl.ds(0, 8), :],
                    rm_hbm.at[pl.ds(0, 8), pl.ds(base, TOK_PER_W)])
    pltpu.sync_copy(rm_v.at[pl.ds(8, 8), :],
                    rm_hbm.at[pl.ds(8, 8), pl.ds(base, TOK_PER_W)])
    pltpu.sync_copy(idx_v, idx_hbm.at[:, pl.ds(base, TOK_PER_W)])


def _sc_route(logits_t):
    mesh = plsc.VectorSubcoreMesh(core_axis_name="c", subcore_axis_name="s")
    return pl.kernel(
        _sc_route_body,
        mesh=mesh,
        out_type=[
            jax.ShapeDtypeStruct((N_EXPERTS, N_TOKENS), jnp.float32),
            jax.ShapeDtypeStruct((TOP_K, N_TOKENS), jnp.int32),
        ],
        scratch_types=[
            pltpu.VMEM((N_EXPERTS, TOK_PER_W), jnp.float32),
            pltpu.VMEM((N_EXPERTS, TOK_PER_W), jnp.float32),
            pltpu.VMEM((TOP_K, TOK_PER_W), jnp.int32),
            pltpu.SemaphoreType.DMA,
        ],
        compiler_params=pltpu.CompilerParams(
            needs_layout_passes=False,
            use_tc_tiling_on_sc=True,
        ),
    )(logits_t)


def kernel(x, W, temperature):
    t = jnp.asarray(temperature, jnp.float32).reshape(1)
    logits_t = _gate_logits_t(x, W, t)
    rm_t, idx_t = _sc_route(logits_t)
    return (rm_t.T, idx_t.T)
